# baseline (device time: 10503 ns/iter reference)
import jax
import jax.numpy as jnp
from jax import lax
from jax.experimental import pallas as pl
from jax.experimental.pallas import tpu as pltpu

N_DEV = 4
GRID = 8


def kernel(x):
    m_per, n = x.shape
    assert m_per % GRID == 0
    m_blk = m_per // GRID

    def body(x_ref, out_ref, acc_ref, comm_ref, send_sems, recv_sems):
        k = pl.program_id(0)
        my_pos = lax.axis_index("i")
        partner = [my_pos ^ 1, 3 - my_pos]
        barrier_sem = pltpu.get_barrier_semaphore()

        @pl.when(k == 0)
        def _():
            for s in (0, 1):
                pl.semaphore_signal(
                    barrier_sem, inc=1,
                    device_id=(partner[s],),
                    device_id_type=pl.DeviceIdType.MESH,
                )

        part = jnp.max(x_ref[:, :], axis=0, keepdims=True)

        @pl.when(k == 0)
        def _():
            acc_ref[:, :] = part

        @pl.when(k > 0)
        def _():
            acc_ref[:, :] = jnp.maximum(acc_ref[:, :], part)

        @pl.when(k == GRID - 1)
        def _():
            pl.semaphore_wait(barrier_sem, 2)

            for s in (0, 1):
                rdma = pltpu.make_async_remote_copy(
                    src_ref=acc_ref,
                    dst_ref=comm_ref.at[s],
                    send_sem=send_sems.at[s],
                    recv_sem=recv_sems.at[s],
                    device_id=(partner[s],),
                    device_id_type=pl.DeviceIdType.MESH,
                )
                rdma.start()
                rdma.wait_recv()
                rdma.wait_send()
                acc_ref[:, :] = jnp.maximum(acc_ref[:, :], comm_ref[s, :, :])

            out_ref[:, :] = acc_ref[:, :]

    return pl.pallas_call(
        body,
        grid=(GRID,),
        out_shape=jax.ShapeDtypeStruct((1, n), x.dtype),
        in_specs=[
            pl.BlockSpec((m_blk, n), lambda k: (k, 0), memory_space=pltpu.VMEM)
        ],
        out_specs=pl.BlockSpec((1, n), lambda k: (0, 0), memory_space=pltpu.VMEM),
        scratch_shapes=[
            pltpu.VMEM((1, n), x.dtype),
            pltpu.VMEM((2, 1, n), x.dtype),
            pltpu.SemaphoreType.DMA((2,)),
            pltpu.SemaphoreType.DMA((2,)),
        ],
        compiler_params=pltpu.CompilerParams(collective_id=0),
    )(x)


# device time: 9278 ns/iter; 1.1320x vs baseline; 1.1320x over previous
import jax
import jax.numpy as jnp
from jax import lax
from jax.experimental import pallas as pl
from jax.experimental.pallas import tpu as pltpu

N_DEV = 4
GRID = 8


def kernel(x):
    m_per, n = x.shape
    assert m_per % GRID == 0
    m_blk = m_per // GRID

    def body(x_ref, out_ref, acc_ref, comm_ref, send_sems, recv_sems):
        k = pl.program_id(0)
        my_pos = lax.axis_index("i")
        barrier_sem = pltpu.get_barrier_semaphore()

        @pl.when(k == 0)
        def _():
            for d in (1, 2, 3):
                pl.semaphore_signal(
                    barrier_sem, inc=1,
                    device_id=((my_pos + d) % N_DEV,),
                    device_id_type=pl.DeviceIdType.MESH,
                )

        part = jnp.max(x_ref[:, :], axis=0, keepdims=True)

        @pl.when(k == 0)
        def _():
            acc_ref[:, :] = part

        @pl.when(k > 0)
        def _():
            acc_ref[:, :] = jnp.maximum(acc_ref[:, :], part)

        @pl.when(k == GRID - 1)
        def _():
            pl.semaphore_wait(barrier_sem, N_DEV - 1)

            rdmas = []
            for d in (1, 2, 3):
                rdma = pltpu.make_async_remote_copy(
                    src_ref=acc_ref,
                    dst_ref=comm_ref.at[d - 1],
                    send_sem=send_sems.at[d - 1],
                    recv_sem=recv_sems.at[d - 1],
                    device_id=((my_pos + d) % N_DEV,),
                    device_id_type=pl.DeviceIdType.MESH,
                )
                rdma.start()
                rdmas.append(rdma)

            acc = acc_ref[:, :]
            for d, rdma in ((1, rdmas[0]), (3, rdmas[2]), (2, rdmas[1])):
                rdma.wait_recv()
                acc = jnp.maximum(acc, comm_ref[d - 1, :, :])
            out_ref[:, :] = acc

            for rdma in rdmas:
                rdma.wait_send()

    return pl.pallas_call(
        body,
        grid=(GRID,),
        out_shape=jax.ShapeDtypeStruct((1, n), x.dtype),
        in_specs=[
            pl.BlockSpec((m_blk, n), lambda k: (k, 0), memory_space=pltpu.VMEM)
        ],
        out_specs=pl.BlockSpec((1, n), lambda k: (0, 0), memory_space=pltpu.VMEM),
        scratch_shapes=[
            pltpu.VMEM((1, n), x.dtype),
            pltpu.VMEM((N_DEV - 1, 1, n), x.dtype),
            pltpu.SemaphoreType.DMA((N_DEV - 1,)),
            pltpu.SemaphoreType.DMA((N_DEV - 1,)),
        ],
        compiler_params=pltpu.CompilerParams(collective_id=0),
    )(x)
